# Initial kernel scaffold; baseline (speedup 1.0000x reference)
import jax
import jax.numpy as jnp
from jax.experimental import pallas as pl


def kernel(x, idx):
    raise NotImplementedError("write your pallas kernel here")


# speedup vs baseline: 82.7883x; 82.7883x over previous
import functools

import jax
import jax.numpy as jnp
from jax import lax
from jax.experimental import pallas as pl
from jax.experimental.pallas import tpu as pltpu
from jax.experimental.pallas import tpu_sc as plsc

_B = 8
_N = 8192
_C, _T, _BR, _S, _SYN = 64, 16, 8, 16, 32
_SEG_T, _BR_T = 8, 4
_NBR = _C * _T * _BR
_NC, _NS, _L = 2, 16, 16
_NW = _NC * _NS
_BRW = _NBR // _NW
_GRP = _BRW // _L
_IDX_PER_GRP = _L * _S * _SYN


def _ha(a, b):
    return a ^ b, a & b


def _fa(a, b, c):
    t = a ^ b
    return t ^ c, (a & b) | (t & c)


def _bs_add(a, b):
    out, c = [], None
    for i in range(len(a)):
        t = a[i] ^ b[i]
        if c is None:
            out.append(t)
            c = a[i] & b[i]
        else:
            out.append(t ^ c)
            c = (a[i] & b[i]) | (t & c)
    out.append(c)
    return out


def _planes(items):
    planes = []
    cur = list(items)
    while cur:
        nxt = []
        while len(cur) >= 3:
            s, cy = _fa(cur.pop(), cur.pop(), cur.pop())
            cur.append(s)
            nxt.append(cy)
        if len(cur) == 2:
            s, cy = _ha(cur.pop(), cur.pop())
            cur.append(s)
            nxt.append(cy)
        planes.append(cur[0])
        cur = nxt
    return planes


@functools.cache
def _build_sc_forward():
    mesh = plsc.VectorSubcoreMesh(core_axis_name="c", subcore_axis_name="s",
                                  num_cores=_NC, num_subcores=_NS)
    return functools.partial(
        pl.kernel,
        out_type=jax.ShapeDtypeStruct((_B * _NBR,), jnp.int32),
        mesh=mesh,
        scratch_types=[
            pltpu.VMEM((_B, _N // _NS), jnp.int32),
            pltpu.VMEM((_N // _NS,), jnp.int32),
            pltpu.VMEM((2 * _N,), jnp.int32),
            pltpu.VMEM((_IDX_PER_GRP,), jnp.int32),
            pltpu.VMEM((_IDX_PER_GRP,), jnp.int32),
            pltpu.VMEM((_B, _BRW), jnp.int32),
            pltpu.VMEM_SHARED((_N,), jnp.int32),
            pltpu.SemaphoreType.DMA,
            pltpu.SemaphoreType.DMA,
        ],
        compiler_params=pltpu.CompilerParams(needs_layout_passes=False),
    )(_sc_forward)


def _sc_forward(x_hbm, idx_hbm, out_hbm, xbuf, mywords, table, slab0, slab1,
                outv, shtab, sem0, sem1):
    sid = lax.axis_index("s")
    cid = lax.axis_index("c")
    wid = sid * _NC + cid

    npack = _N // _NS
    base_n = sid * npack
    for b in range(_B):
        pltpu.sync_copy(x_hbm.at[pl.ds(b * _N + base_n, npack)], xbuf.at[b])
    for v in range(npack // _L):
        w = xbuf[0, pl.ds(v * _L, _L)]
        for b in range(1, _B):
            w = w | (xbuf[b, pl.ds(v * _L, _L)] << b)
        mywords[pl.ds(v * _L, _L)] = w
    pltpu.sync_copy(mywords, shtab.at[pl.ds(base_n, npack)])
    plsc.subcore_barrier()
    pltpu.sync_copy(shtab, table.at[pl.ds(0, _N)])
    table[pl.ds(2 * _N - _L, _L)] = jnp.zeros((_L,), jnp.int32)

    lane512 = lax.broadcasted_iota(jnp.int32, (_L,), 0) * (_S * _SYN)
    base_idx = wid * _BRW * _S * _SYN

    def compute(g, slab_ref):
        quad_segons = []
        for q in range(4):
            words = []
            for k in range(_SYN):
                cmb = None
                for j in range(4):
                    s = q * 4 + j
                    iv = lane512 + (s * _SYN + k)
                    idxv = plsc.load_gather(slab_ref, [iv])
                    w = plsc.load_gather(table, [idxv & (2 * _N - 1)])
                    if j:
                        w = w << (8 * j)
                    cmb = w if cmb is None else (cmb | w)
                words.append(cmb)
            p = _planes(words)
            quad_segons.append(p[3] | p[4] | p[5])
        u = _planes(quad_segons)
        subs = [[pl >> (8 * sub) for pl in u] for sub in range(4)]
        s1 = _bs_add(subs[0], subs[1])
        s2 = _bs_add(subs[2], subs[3])
        sm = _bs_add(s1, s2)
        on = sm[2] | sm[3] | sm[4]
        for b in range(_B):
            outv[b, pl.ds(g * _L, _L)] = (on >> b) & 1

    pltpu.async_copy(idx_hbm.at[pl.ds(base_idx, _IDX_PER_GRP)], slab0, sem0)

    def pair_body(i, carry):
        g0 = 2 * i
        pltpu.async_copy(
            idx_hbm.at[pl.ds(base_idx + (g0 + 1) * _IDX_PER_GRP,
                             _IDX_PER_GRP)], slab1, sem1)
        pltpu.make_async_copy(idx_hbm.at[pl.ds(0, _IDX_PER_GRP)], slab0,
                              sem0).wait()
        compute(g0, slab0)

        @pl.when(i < _GRP // 2 - 1)
        def _prefetch_next():
            pltpu.async_copy(
                idx_hbm.at[pl.ds(base_idx + (g0 + 2) * _IDX_PER_GRP,
                                 _IDX_PER_GRP)], slab0, sem0)

        pltpu.make_async_copy(idx_hbm.at[pl.ds(0, _IDX_PER_GRP)], slab1,
                              sem1).wait()
        compute(g0 + 1, slab1)
        return carry

    lax.fori_loop(0, _GRP // 2, pair_body, 0)
    for b in range(_B):
        pltpu.sync_copy(outv.at[b],
                        out_hbm.at[pl.ds(b * _NBR + wid * _BRW, _BRW)])


def kernel(x, idx):
    bits = _build_sc_forward()(x.reshape(-1), idx.reshape(-1))
    prev = x.astype(jnp.bool_)
    bmat = bits.reshape(_B, _C, _T, _BR)
    branch_on = bmat.astype(jnp.bool_)
    final = bmat[:, :, 0]
    return (final, prev, branch_on)


# speedup vs baseline: 153.6672x; 1.8561x over previous
import functools

import jax
import jax.numpy as jnp
from jax import lax
from jax.experimental import pallas as pl
from jax.experimental.pallas import tpu as pltpu
from jax.experimental.pallas import tpu_sc as plsc

_B = 8
_N = 8192
_C, _T, _BR, _S, _SYN = 64, 16, 8, 16, 32
_SEG_T, _BR_T = 8, 4
_TB = _T * _BR
_NBR = _C * _TB
_NC, _NS, _L = 2, 16, 16
_NW = _NC * _NS
_TBW = _TB // _NW
_ROWS = _S * _SYN
_QROWS = _ROWS // 4


def _bs_add(a, b):
    out, c = [], None
    for i in range(len(a)):
        t = a[i] ^ b[i]
        if c is None:
            out.append(t)
            c = a[i] & b[i]
        else:
            out.append(t ^ c)
            c = (a[i] & b[i]) | (t & c)
    out.append(c)
    return out


def _planes(items):
    planes = []
    cur = list(items)
    while cur:
        nxt = []
        while len(cur) >= 3:
            s, cy = _fa(cur.pop(), cur.pop(), cur.pop())
            cur.append(s)
            nxt.append(cy)
        if len(cur) == 2:
            s, cy = _ha(cur.pop(), cur.pop())
            cur.append(s)
            nxt.append(cy)
        planes.append(cur[0])
        cur = nxt
    return planes


def _ha(a, b):
    return a ^ b, a & b


def _fa(a, b, c):
    t = a ^ b
    return t ^ c, (a & b) | (t & c)


@functools.cache
def _build_sc_forward():
    mesh = plsc.VectorSubcoreMesh(core_axis_name="c", subcore_axis_name="s",
                                  num_cores=_NC, num_subcores=_NS)
    return functools.partial(
        pl.kernel,
        out_type=jax.ShapeDtypeStruct((_B * _NBR,), jnp.int32),
        mesh=mesh,
        scratch_types=[
            pltpu.VMEM((_N // _NS * _B,), jnp.int32),
            pltpu.VMEM((_N // _NS,), jnp.int32),
            pltpu.VMEM((_N + _L,), jnp.int32),
            pltpu.VMEM((_QROWS // 8, 8, _C), jnp.int32),
            pltpu.VMEM((_QROWS // 8, 8, _C), jnp.int32),
            pltpu.VMEM((_B * _TBW * _C,), jnp.int32),
            pltpu.VMEM_SHARED((_N,), jnp.int32),
            pltpu.SemaphoreType.DMA,
            pltpu.SemaphoreType.DMA,
        ],
        compiler_params=pltpu.CompilerParams(needs_layout_passes=False,
                                             use_tc_tiling_on_sc=True),
    )(_sc_forward)


def _sc_forward(x_hbm, idx_hbm, out_hbm, xbuf, mywords, table, slab0, slab1,
                outv, shtab, sem0, sem1):
    sid = lax.axis_index("s")
    cid = lax.axis_index("c")
    wid = sid * _NC + cid

    npack = _N // _NS
    pltpu.sync_copy(x_hbm.at[pl.ds(sid * (npack * _B), npack * _B)], xbuf)
    for ti in range(4):
        for j in range(8):
            w = xbuf[pl.ds(ti * 1024 + j * _L, _L)]
            for b in range(1, _B):
                w = w | (xbuf[pl.ds(ti * 1024 + b * 128 + j * _L, _L)] << b)
            mywords[pl.ds(ti * 128 + j * _L, _L)] = w
    pltpu.sync_copy(mywords, shtab.at[pl.ds(sid * npack, npack)])
    plsc.subcore_barrier()
    pltpu.sync_copy(shtab, table.at[pl.ds(0, _N)])
    table[pl.ds(_N, _L)] = jnp.zeros((_L,), jnp.int32)

    lane = lax.broadcasted_iota(jnp.int32, (_L,), 0)
    cvs = [lane + cq * _L for cq in range(4)]
    row_base = wid * _TBW * _ROWS

    def chunk_quads(slab_ref, qs_acc):
        for cq in range(4):
            words = []
            for k in range(_SYN):
                cmb = None
                for j in range(4):
                    tvec = jnp.full((_L,), j * 4 + k // 8, jnp.int32)
                    rvec = jnp.full((_L,), k % 8, jnp.int32)
                    idxv = plsc.load_gather(slab_ref,
                                            [tvec, rvec, cvs[cq]])
                    safe = jnp.minimum(idxv & 0x3FFF, _N)
                    w = plsc.load_gather(table, [safe])
                    if j:
                        w = w << (8 * j)
                    cmb = w if cmb is None else (cmb | w)
                words.append(cmb)
            p = _planes(words)
            qs_acc[cq].append(p[3] | p[4] | p[5])

    nqt = _QROWS // 8

    def add_chunk(sl, carry):
        qs = [[] for _ in range(4)]
        chunk_quads(sl, qs)
        out = []
        for cq in range(4):
            c0, c1, c2 = carry[3 * cq:3 * cq + 3]
            v = qs[cq][0]
            t0 = c0 & v
            c0 = c0 ^ v
            t1 = c1 & t0
            c1 = c1 ^ t0
            c2 = c2 | t1
            out.extend((c0, c1, c2))
        return tuple(out)

    def pair_body(p, carry):
        qi0 = 2 * p
        pltpu.make_async_copy(idx_hbm.at[pl.ds(0, nqt), :, :], slab0,
                              sem0).wait()
        carry = add_chunk(slab0, carry)

        @pl.when(qi0 + 2 < 4 * _TBW)
        def _prefetch0():
            t0 = row_base // 8 + (qi0 + 2) * nqt
            pltpu.async_copy(idx_hbm.at[pl.ds(t0, nqt), :, :], slab0, sem0)

        pltpu.make_async_copy(idx_hbm.at[pl.ds(0, nqt), :, :], slab1,
                              sem1).wait()
        carry = add_chunk(slab1, carry)

        @pl.when(qi0 + 3 < 4 * _TBW)
        def _prefetch1():
            t1 = row_base // 8 + (qi0 + 3) * nqt
            pltpu.async_copy(idx_hbm.at[pl.ds(t1, nqt), :, :], slab1, sem1)

        finish = (p % 2) == 1
        tb = p // 2

        @pl.when(finish)
        def _reduce_store():
            for cq in range(4):
                u = carry[3 * cq:3 * cq + 3]
                subs = [[pl_ >> (8 * sub) for pl_ in u] for sub in range(4)]
                s1 = _bs_add(subs[0], subs[1])
                s2 = _bs_add(subs[2], subs[3])
                sm = _bs_add(s1, s2)
                on = sm[2] | sm[3] | sm[4]
                for b in range(_B):
                    outv[pl.ds(b * (_TBW * _C) + tb * _C + cq * _L, _L)] = (
                        (on >> b) & 1)

        zero = jnp.zeros((_L,), jnp.int32)
        carry = tuple(jnp.where(finish, zero, c) for c in carry)
        return carry

    pltpu.async_copy(idx_hbm.at[pl.ds(row_base // 8, nqt), :, :],
                     slab0, sem0)
    pltpu.async_copy(idx_hbm.at[pl.ds(row_base // 8 + nqt, nqt), :, :],
                     slab1, sem1)
    zero = jnp.zeros((_L,), jnp.int32)
    lax.fori_loop(0, 2 * _TBW, pair_body, tuple(zero for _ in range(12)))
    for b in range(_B):
        pltpu.sync_copy(outv.at[pl.ds(b * (_TBW * _C), _TBW * _C)],
                        out_hbm.at[pl.ds(b * _NBR + wid * (_TBW * _C),
                                         _TBW * _C)])


def kernel(x, idx):
    x1 = x.reshape(_B, _N // 128, 128).transpose(1, 0, 2).reshape(-1)
    idx2 = jnp.moveaxis(idx, 0, -1).reshape(_TB * _ROWS // 8, 8, _C)
    bits = _build_sc_forward()(x1, idx2)
    prev = x.astype(jnp.bool_)
    bmat = bits.reshape(_B, _T, _BR, _C)
    branch_on = bmat.transpose(0, 3, 1, 2).astype(jnp.bool_)
    final = bmat[:, 0].transpose(0, 2, 1)
    return (final, prev, branch_on)


# speedup vs baseline: 166.6346x; 1.0844x over previous
import functools

import jax
import jax.numpy as jnp
from jax import lax
from jax.experimental import pallas as pl
from jax.experimental.pallas import tpu as pltpu
from jax.experimental.pallas import tpu_sc as plsc

_B = 8
_N = 8192
_C, _T, _BR, _S, _SYN = 64, 16, 8, 16, 32
_SEG_T, _BR_T = 8, 4
_TB = _T * _BR
_NBR = _C * _TB
_NC, _NS, _L = 2, 16, 16
_NW = _NC * _NS
_TBW = _TB // _NW
_ROWS = _S * _SYN
_QROWS = _ROWS // 4


def _bs_add(a, b):
    out, c = [], None
    for i in range(len(a)):
        t = a[i] ^ b[i]
        if c is None:
            out.append(t)
            c = a[i] & b[i]
        else:
            out.append(t ^ c)
            c = (a[i] & b[i]) | (t & c)
    out.append(c)
    return out


def _planes(items):
    planes = []
    cur = list(items)
    while cur:
        nxt = []
        while len(cur) >= 3:
            s, cy = _fa(cur.pop(), cur.pop(), cur.pop())
            cur.append(s)
            nxt.append(cy)
        if len(cur) == 2:
            s, cy = _ha(cur.pop(), cur.pop())
            cur.append(s)
            nxt.append(cy)
        planes.append(cur[0])
        cur = nxt
    return planes


def _ha(a, b):
    return a ^ b, a & b


def _fa(a, b, c):
    t = a ^ b
    return t ^ c, (a & b) | (t & c)


@functools.cache
def _build_sc_forward():
    mesh = plsc.VectorSubcoreMesh(core_axis_name="c", subcore_axis_name="s",
                                  num_cores=_NC, num_subcores=_NS)
    return functools.partial(
        pl.kernel,
        out_type=jax.ShapeDtypeStruct((_B * _NBR,), jnp.int32),
        mesh=mesh,
        scratch_types=[
            pltpu.VMEM((_N // _NS * _B,), jnp.int32),
            pltpu.VMEM((_N // _NS,), jnp.int32),
            pltpu.VMEM((_N + _L,), jnp.int32),
            pltpu.VMEM((_QROWS // 8, 8, _C), jnp.int32),
            pltpu.VMEM((_QROWS // 8, 8, _C), jnp.int32),
            pltpu.VMEM((_B * _TBW * _C,), jnp.int32),
            pltpu.VMEM_SHARED((_N,), jnp.int32),
            pltpu.SemaphoreType.DMA,
            pltpu.SemaphoreType.DMA,
        ],
        compiler_params=pltpu.CompilerParams(needs_layout_passes=False,
                                             use_tc_tiling_on_sc=True),
    )(_sc_forward)


def _sc_forward(x_hbm, idx_hbm, out_hbm, xbuf, mywords, table, slab0, slab1,
                outv, shtab, sem0, sem1):
    sid = lax.axis_index("s")
    cid = lax.axis_index("c")
    wid = sid * _NC + cid

    row_base = wid * _TBW * _ROWS
    nqt = _QROWS // 8
    pltpu.async_copy(idx_hbm.at[pl.ds(row_base // 8, nqt), :, :],
                     slab0, sem0)
    pltpu.async_copy(idx_hbm.at[pl.ds(row_base // 8 + nqt, nqt), :, :],
                     slab1, sem1)

    npack = _N // _NS
    pltpu.sync_copy(x_hbm.at[pl.ds(sid * (npack * _B), npack * _B)], xbuf)
    for ti in range(4):
        for j in range(8):
            w = xbuf[pl.ds(ti * 1024 + j * _L, _L)]
            for b in range(1, _B):
                w = w | (xbuf[pl.ds(ti * 1024 + b * 128 + j * _L, _L)] << b)
            mywords[pl.ds(ti * 128 + j * _L, _L)] = w
    pltpu.sync_copy(mywords, shtab.at[pl.ds(sid * npack, npack)])
    plsc.subcore_barrier()
    pltpu.sync_copy(shtab, table.at[pl.ds(0, _N)])
    table[pl.ds(_N, _L)] = jnp.zeros((_L,), jnp.int32)

    lane = lax.broadcasted_iota(jnp.int32, (_L,), 0)
    cvs = [lane + cq * _L for cq in range(4)]

    def chunk_quads(slab_ref, qs_acc):
        for cq in range(4):
            words = []
            for k in range(_SYN):
                cmb = None
                for j in range(4):
                    tvec = jnp.full((_L,), j * 4 + k // 8, jnp.int32)
                    rvec = jnp.full((_L,), k % 8, jnp.int32)
                    idxv = plsc.load_gather(slab_ref,
                                            [tvec, rvec, cvs[cq]])
                    safe = jnp.minimum(idxv & 0x3FFF, _N)
                    w = plsc.load_gather(table, [safe])
                    if j:
                        w = w << (8 * j)
                    cmb = w if cmb is None else (cmb | w)
                words.append(cmb)
            p = _planes(words)
            qs_acc[cq].append(p[3] | p[4] | p[5])

    def add_chunk(sl, carry):
        qs = [[] for _ in range(4)]
        chunk_quads(sl, qs)
        out = []
        for cq in range(4):
            c0, c1, c2 = carry[3 * cq:3 * cq + 3]
            v = qs[cq][0]
            t0 = c0 & v
            c0 = c0 ^ v
            t1 = c1 & t0
            c1 = c1 ^ t0
            c2 = c2 | t1
            out.extend((c0, c1, c2))
        return tuple(out)

    def pair_body(p, carry):
        qi0 = 2 * p
        pltpu.make_async_copy(idx_hbm.at[pl.ds(0, nqt), :, :], slab0,
                              sem0).wait()
        carry = add_chunk(slab0, carry)

        @pl.when(qi0 + 2 < 4 * _TBW)
        def _prefetch0():
            t0 = row_base // 8 + (qi0 + 2) * nqt
            pltpu.async_copy(idx_hbm.at[pl.ds(t0, nqt), :, :], slab0, sem0)

        pltpu.make_async_copy(idx_hbm.at[pl.ds(0, nqt), :, :], slab1,
                              sem1).wait()
        carry = add_chunk(slab1, carry)

        @pl.when(qi0 + 3 < 4 * _TBW)
        def _prefetch1():
            t1 = row_base // 8 + (qi0 + 3) * nqt
            pltpu.async_copy(idx_hbm.at[pl.ds(t1, nqt), :, :], slab1, sem1)

        finish = (p % 2) == 1
        tb = p // 2

        @pl.when(finish)
        def _reduce_store():
            for cq in range(4):
                u = carry[3 * cq:3 * cq + 3]
                subs = [[pl_ >> (8 * sub) for pl_ in u] for sub in range(4)]
                s1 = _bs_add(subs[0], subs[1])
                s2 = _bs_add(subs[2], subs[3])
                sm = _bs_add(s1, s2)
                on = sm[2] | sm[3] | sm[4]
                for b in range(_B):
                    outv[pl.ds(b * (_TBW * _C) + tb * _C + cq * _L, _L)] = (
                        (on >> b) & 1)

        zero = jnp.zeros((_L,), jnp.int32)
        carry = tuple(jnp.where(finish, zero, c) for c in carry)
        return carry

    zero = jnp.zeros((_L,), jnp.int32)
    lax.fori_loop(0, 2 * _TBW, pair_body, tuple(zero for _ in range(12)))
    for b in range(_B):
        pltpu.sync_copy(outv.at[pl.ds(b * (_TBW * _C), _TBW * _C)],
                        out_hbm.at[pl.ds(b * _NBR + wid * (_TBW * _C),
                                         _TBW * _C)])


def kernel(x, idx):
    x1 = x.reshape(_B, _N // 128, 128).transpose(1, 0, 2).reshape(-1)
    idx2 = jnp.moveaxis(idx, 0, -1).reshape(_TB * _ROWS // 8, 8, _C)
    bits = _build_sc_forward()(x1, idx2)
    prev = x.astype(jnp.bool_)
    bmat = bits.reshape(_B, _T, _BR, _C)
    branch_on = bmat.transpose(0, 3, 1, 2).astype(jnp.bool_)
    final = bmat[:, 0].transpose(0, 2, 1)
    return (final, prev, branch_on)
